# Initial kernel scaffold; baseline (speedup 1.0000x reference)
import jax
import jax.numpy as jnp
from jax.experimental import pallas as pl


def kernel(cls_out0, cls_out1, cls_out2, cls_out3, cls_out4, reg_out0, reg_out1, reg_out2, reg_out3, reg_out4, img_h, img_w):
    raise NotImplementedError("write your pallas kernel here")


# speedup vs baseline: 9.0841x; 9.0841x over previous
import functools
import numpy as np
import jax
import jax.numpy as jnp
from jax import lax
from jax.experimental import pallas as pl
from jax.experimental.pallas import tpu as pltpu

_STRIDES = (4, 8, 16, 32, 64)
_SCALES = (8.0,)
_RATIOS = (0.5, 1.0, 2.0)
_A = len(_SCALES) * len(_RATIOS)
_PRE_NMS = 1000
_MAX_PER_IMG = 500
_NMS_THR = 0.7
_FEAT_HW = [(128, 128), (64, 64), (32, 32), (16, 16), (8, 8)]

_N_PAD = 4096
_BLK = 128
_NBLK = _N_PAD // _BLK


def _np_grid_anchors(H, W, stride):
    scales = np.asarray(_SCALES, np.float32)
    ratios = np.asarray(_RATIOS, np.float32)
    h_r = np.sqrt(ratios)
    w_r = (1.0 / h_r).astype(np.float32)
    ws = (np.float32(stride) * w_r[:, None] * scales[None, :]).reshape(-1)
    hs = (np.float32(stride) * h_r[:, None] * scales[None, :]).reshape(-1)
    base = np.stack([-0.5 * ws, -0.5 * hs, 0.5 * ws, 0.5 * hs], axis=1)
    xs = np.arange(W, dtype=np.float32) * np.float32(stride)
    ys = np.arange(H, dtype=np.float32) * np.float32(stride)
    sx, sy = np.meshgrid(xs, ys)
    shifts = np.stack([sx, sy, sx, sy], axis=-1).reshape(-1, 4)
    return (shifts[:, None, :] + base[None, :, :]).reshape(-1, 4).astype(np.float32)


_ANCHORS = [_np_grid_anchors(H, W, s) for (H, W), s in zip(_FEAT_HW, _STRIDES)]
_KS = [min(_PRE_NMS, H * W * _A) for (H, W) in _FEAT_HW]
_NTOT = sum(_KS)
_LEVELS = np.concatenate(
    [np.full((k,), float(l), np.float32) for l, k in enumerate(_KS)]
    + [np.zeros((_N_PAD - _NTOT,), np.float32)]
)


def _decode_kernel(a_ref, d_ref, lg_ref, lv_ref, ih_ref, iw_ref,
                   prop_ref, boff_ref, sc_ref):
    ax1 = a_ref[0, 0:1, :]
    ay1 = a_ref[0, 1:2, :]
    ax2 = a_ref[0, 2:3, :]
    ay2 = a_ref[0, 3:4, :]
    dx = d_ref[0, 0:1, :]
    dy = d_ref[0, 1:2, :]
    dw = d_ref[0, 2:3, :]
    dh = d_ref[0, 3:4, :]
    px = (ax1 + ax2) * 0.5
    py = (ay1 + ay2) * 0.5
    pw = ax2 - ax1
    ph = ay2 - ay1
    max_ratio = np.float32(np.log(1000.0 / 16.0))
    dw = jnp.clip(dw, -max_ratio, max_ratio)
    dh = jnp.clip(dh, -max_ratio, max_ratio)
    gx = px + pw * dx
    gy = py + ph * dy
    gw = pw * jnp.exp(dw)
    gh = ph * jnp.exp(dh)
    ihv = ih_ref[0:1, 0:1]
    iwv = iw_ref[0:1, 0:1]
    zero = jnp.float32(0.0)
    x1 = jnp.minimum(jnp.maximum(gx - 0.5 * gw, zero), iwv)
    y1 = jnp.minimum(jnp.maximum(gy - 0.5 * gh, zero), ihv)
    x2 = jnp.minimum(jnp.maximum(gx + 0.5 * gw, zero), iwv)
    y2 = jnp.minimum(jnp.maximum(gy + 0.5 * gh, zero), ihv)
    prop_ref[0, 0:1, :] = x1
    prop_ref[0, 1:2, :] = y1
    prop_ref[0, 2:3, :] = x2
    prop_ref[0, 3:4, :] = y2
    w = x2 - x1
    h = y2 - y1
    score = jax.nn.sigmoid(lg_ref[0, 0:1, :])
    score = jnp.where((w > zero) & (h > zero), score, jnp.float32(-1.0))
    sc_ref[0, 0:1, :] = score
    off = lv_ref[0, 0:1, :] * (jnp.maximum(ihv, iwv) + 1.0)
    boff_ref[0, 0:1, :] = x1 + off
    boff_ref[0, 1:2, :] = y1 + off
    boff_ref[0, 2:3, :] = x2 + off
    boff_ref[0, 3:4, :] = y2 + off


def _nms_kernel(bt_ref, bc_ref, s_ref, out_ref, slab_ref):
    bx1 = bt_ref[0, 0:1, :]
    by1 = bt_ref[0, 1:2, :]
    bx2 = bt_ref[0, 2:3, :]
    by2 = bt_ref[0, 3:4, :]
    area_b = (bx2 - bx1) * (by2 - by1)
    jidx = lax.broadcasted_iota(jnp.int32, (1, _N_PAD), 1)
    thr = jnp.float32(_NMS_THR)
    eps = jnp.float32(1e-9)

    def outer(r, supp):
        base = pl.multiple_of(r * _BLK, _BLK)
        ax1 = bc_ref[0, pl.ds(base, _BLK), 0:1]
        ay1 = bc_ref[0, pl.ds(base, _BLK), 1:2]
        ax2 = bc_ref[0, pl.ds(base, _BLK), 2:3]
        ay2 = bc_ref[0, pl.ds(base, _BLK), 3:4]
        area_a = (ax2 - ax1) * (ay2 - ay1)
        ltx = jnp.maximum(ax1, bx1)
        lty = jnp.maximum(ay1, by1)
        rbx = jnp.minimum(ax2, bx2)
        rby = jnp.minimum(ay2, by2)
        zero = jnp.float32(0.0)
        inter = jnp.maximum(rbx - ltx, zero) * jnp.maximum(rby - lty, zero)
        union = area_a + area_b - inter + eps
        rowg = base + lax.broadcasted_iota(jnp.int32, (_BLK, _N_PAD), 0)
        colg = lax.broadcasted_iota(jnp.int32, (_BLK, _N_PAD), 1)
        sup = (inter > thr * union) & (colg > rowg)
        slab_ref[:, :] = sup.astype(jnp.float32)
        for i in range(_BLK):
            g = base + i
            row = slab_ref[i:i + 1, :]
            onehot = (jidx == g).astype(jnp.float32)
            sflag = jnp.sum(supp * onehot)
            gate = 1.0 - sflag
            supp = jnp.maximum(supp, row * gate)
        return supp

    supp = lax.fori_loop(0, _NBLK, outer, jnp.zeros((1, _N_PAD), jnp.float32))
    keep = supp < 0.5
    out_ref[0, 0:1, :] = jnp.where(keep, s_ref[0, 0:1, :], jnp.float32(-1.0))


def _run_decode(anchors_t, deltas_t, logits, levels, ih, iw):
    B = anchors_t.shape[0]
    spec4 = pl.BlockSpec((1, 4, _N_PAD), lambda b: (b, 0, 0))
    spec1 = pl.BlockSpec((1, 1, _N_PAD), lambda b: (b, 0, 0))
    specl = pl.BlockSpec((1, 1, _N_PAD), lambda b: (0, 0, 0))
    specs = pl.BlockSpec((1, 1), lambda b: (0, 0))
    return pl.pallas_call(
        _decode_kernel,
        grid=(B,),
        in_specs=[spec4, spec4, spec1, specl, specs, specs],
        out_specs=[spec4, spec4, spec1],
        out_shape=[
            jax.ShapeDtypeStruct((B, 4, _N_PAD), jnp.float32),
            jax.ShapeDtypeStruct((B, 4, _N_PAD), jnp.float32),
            jax.ShapeDtypeStruct((B, 1, _N_PAD), jnp.float32),
        ],
    )(anchors_t, deltas_t, logits, levels, ih, iw)


def _run_nms(b_st, b_cols, s_s):
    B = b_st.shape[0]
    spec4 = pl.BlockSpec((1, 4, _N_PAD), lambda b: (b, 0, 0))
    specc = pl.BlockSpec((1, _N_PAD, 4), lambda b: (b, 0, 0))
    spec1 = pl.BlockSpec((1, 1, _N_PAD), lambda b: (b, 0, 0))
    return pl.pallas_call(
        _nms_kernel,
        grid=(B,),
        in_specs=[spec4, specc, spec1],
        out_specs=spec1,
        out_shape=jax.ShapeDtypeStruct((B, 1, _N_PAD), jnp.float32),
        scratch_shapes=[pltpu.VMEM((_BLK, _N_PAD), jnp.float32)],
    )(b_st, b_cols, s_s)


@jax.jit
def _pipeline(cls_outs, reg_outs, img_h, img_w):
    B = cls_outs[0].shape[0]
    lg_all, dl_all, an_all = [], [], []
    for lvl, (c, r) in enumerate(zip(cls_outs, reg_outs)):
        H, W = _FEAT_HW[lvl]
        n = H * W * _A
        k = _KS[lvl]
        logits = jnp.transpose(c, (0, 2, 3, 1)).reshape(B, n)
        rg = jnp.transpose(r.reshape(B, _A, 4, H, W), (0, 3, 4, 1, 2)).reshape(B, n, 4)
        vals, inds = lax.top_k(logits, k)
        dl = jnp.take_along_axis(rg, inds[:, :, None], axis=1)
        an = jnp.asarray(_ANCHORS[lvl])[inds]
        lg_all.append(vals)
        dl_all.append(dl)
        an_all.append(an)
    logits = jnp.concatenate(lg_all, axis=1)
    deltas = jnp.concatenate(dl_all, axis=1)
    anchors = jnp.concatenate(an_all, axis=1)
    pad = _N_PAD - _NTOT
    logits = jnp.pad(logits, ((0, 0), (0, pad)), constant_values=-1e30)
    deltas = jnp.pad(deltas, ((0, 0), (0, pad), (0, 0)))
    anchors = jnp.pad(anchors, ((0, 0), (0, pad), (0, 0)))
    anchors_t = jnp.transpose(anchors, (0, 2, 1))
    deltas_t = jnp.transpose(deltas, (0, 2, 1))
    logits3 = logits[:, None, :]
    levels3 = jnp.asarray(_LEVELS)[None, None, :]
    ih = jnp.asarray(img_h).astype(jnp.float32).reshape(1, 1)
    iw = jnp.asarray(img_w).astype(jnp.float32).reshape(1, 1)

    props, boff, scores = _run_decode(anchors_t, deltas_t, logits3, levels3, ih, iw)
    scores2d = scores[:, 0, :]

    order = jnp.argsort(-scores2d, axis=1)
    s_s = jnp.take_along_axis(scores2d, order, axis=1)
    p_st = jnp.take_along_axis(props, order[:, None, :], axis=2)
    b_st = jnp.take_along_axis(boff, order[:, None, :], axis=2)
    b_cols = jnp.transpose(b_st, (0, 2, 1))

    final = _run_nms(b_st, b_cols, s_s[:, None, :])[:, 0, :]

    vals, sel = lax.top_k(final, _MAX_PER_IMG)
    good = vals > 0.0
    p_cols = jnp.transpose(p_st, (0, 2, 1))
    sel_boxes = jnp.take_along_axis(p_cols, sel[:, :, None], axis=1)
    out_boxes = jnp.where(good[:, :, None], sel_boxes, 0.0)
    out_scores = jnp.where(good, vals, 0.0)
    return out_boxes, out_scores


def kernel(cls_out0, cls_out1, cls_out2, cls_out3, cls_out4,
           reg_out0, reg_out1, reg_out2, reg_out3, reg_out4, img_h, img_w):
    return _pipeline(
        [cls_out0, cls_out1, cls_out2, cls_out3, cls_out4],
        [reg_out0, reg_out1, reg_out2, reg_out3, reg_out4],
        img_h, img_w)


# speedup vs baseline: 12.4154x; 1.3667x over previous
import functools
import numpy as np
import jax
import jax.numpy as jnp
from jax import lax
from jax.experimental import pallas as pl
from jax.experimental.pallas import tpu as pltpu

_STRIDES = (4, 8, 16, 32, 64)
_SCALES = (8.0,)
_RATIOS = (0.5, 1.0, 2.0)
_A = len(_SCALES) * len(_RATIOS)
_PRE_NMS = 1000
_MAX_PER_IMG = 500
_NMS_THR = 0.7
_FEAT_HW = [(128, 128), (64, 64), (32, 32), (16, 16), (8, 8)]

_N_PAD = 4096
_BLK = 128
_NBLK = _N_PAD // _BLK


def _np_grid_anchors(H, W, stride):
    scales = np.asarray(_SCALES, np.float32)
    ratios = np.asarray(_RATIOS, np.float32)
    h_r = np.sqrt(ratios)
    w_r = (1.0 / h_r).astype(np.float32)
    ws = (np.float32(stride) * w_r[:, None] * scales[None, :]).reshape(-1)
    hs = (np.float32(stride) * h_r[:, None] * scales[None, :]).reshape(-1)
    base = np.stack([-0.5 * ws, -0.5 * hs, 0.5 * ws, 0.5 * hs], axis=1)
    xs = np.arange(W, dtype=np.float32) * np.float32(stride)
    ys = np.arange(H, dtype=np.float32) * np.float32(stride)
    sx, sy = np.meshgrid(xs, ys)
    shifts = np.stack([sx, sy, sx, sy], axis=-1).reshape(-1, 4)
    return (shifts[:, None, :] + base[None, :, :]).reshape(-1, 4).astype(np.float32)


_ANCHORS = [_np_grid_anchors(H, W, s) for (H, W), s in zip(_FEAT_HW, _STRIDES)]
_KS = [min(_PRE_NMS, H * W * _A) for (H, W) in _FEAT_HW]
_NTOT = sum(_KS)
_LEVELS = np.concatenate(
    [np.full((k,), float(l), np.float32) for l, k in enumerate(_KS)]
    + [np.zeros((_N_PAD - _NTOT,), np.float32)]
)


def _decode_kernel(a_ref, d_ref, lg_ref, lv_ref, ih_ref, iw_ref,
                   prop_ref, boff_ref, sc_ref):
    ax1 = a_ref[0, 0:1, :]
    ay1 = a_ref[0, 1:2, :]
    ax2 = a_ref[0, 2:3, :]
    ay2 = a_ref[0, 3:4, :]
    dx = d_ref[0, 0:1, :]
    dy = d_ref[0, 1:2, :]
    dw = d_ref[0, 2:3, :]
    dh = d_ref[0, 3:4, :]
    px = (ax1 + ax2) * 0.5
    py = (ay1 + ay2) * 0.5
    pw = ax2 - ax1
    ph = ay2 - ay1
    max_ratio = np.float32(np.log(1000.0 / 16.0))
    dw = jnp.clip(dw, -max_ratio, max_ratio)
    dh = jnp.clip(dh, -max_ratio, max_ratio)
    gx = px + pw * dx
    gy = py + ph * dy
    gw = pw * jnp.exp(dw)
    gh = ph * jnp.exp(dh)
    ihv = ih_ref[0:1, 0:1]
    iwv = iw_ref[0:1, 0:1]
    zero = jnp.float32(0.0)
    x1 = jnp.minimum(jnp.maximum(gx - 0.5 * gw, zero), iwv)
    y1 = jnp.minimum(jnp.maximum(gy - 0.5 * gh, zero), ihv)
    x2 = jnp.minimum(jnp.maximum(gx + 0.5 * gw, zero), iwv)
    y2 = jnp.minimum(jnp.maximum(gy + 0.5 * gh, zero), ihv)
    prop_ref[0, 0:1, :] = x1
    prop_ref[0, 1:2, :] = y1
    prop_ref[0, 2:3, :] = x2
    prop_ref[0, 3:4, :] = y2
    w = x2 - x1
    h = y2 - y1
    score = jax.nn.sigmoid(lg_ref[0, 0:1, :])
    score = jnp.where((w > zero) & (h > zero), score, jnp.float32(-1.0))
    sc_ref[0, 0:1, :] = score
    off = lv_ref[0, 0:1, :] * (jnp.maximum(ihv, iwv) + 1.0)
    boff_ref[0, 0:1, :] = x1 + off
    boff_ref[0, 1:2, :] = y1 + off
    boff_ref[0, 2:3, :] = x2 + off
    boff_ref[0, 3:4, :] = y2 + off


def _nms_kernel(bt_ref, bc_ref, s_ref, out_ref,
                slab_ref, diag_ref, gcol_ref, supp_ref):
    bx1 = bt_ref[0, 0:1, :]
    by1 = bt_ref[0, 1:2, :]
    bx2 = bt_ref[0, 2:3, :]
    by2 = bt_ref[0, 3:4, :]
    area_b = (bx2 - bx1) * (by2 - by1)
    thr = jnp.float32(_NMS_THR)
    eps = jnp.float32(1e-9)
    zero = jnp.float32(0.0)
    supp_ref[:, :] = jnp.zeros((1, _N_PAD), jnp.float32)
    jidx_blk = lax.broadcasted_iota(jnp.int32, (1, _BLK), 1)
    tri_blk = (lax.broadcasted_iota(jnp.int32, (_BLK, _BLK), 1)
               > lax.broadcasted_iota(jnp.int32, (_BLK, _BLK), 0))

    def outer(r, _):
        base = pl.multiple_of(r * _BLK, _BLK)
        ax1 = bc_ref[0, pl.ds(base, _BLK), 0:1]
        ay1 = bc_ref[0, pl.ds(base, _BLK), 1:2]
        ax2 = bc_ref[0, pl.ds(base, _BLK), 2:3]
        ay2 = bc_ref[0, pl.ds(base, _BLK), 3:4]
        area_a = (ax2 - ax1) * (ay2 - ay1)
        ltx = jnp.maximum(ax1, bx1)
        lty = jnp.maximum(ay1, by1)
        rbx = jnp.minimum(ax2, bx2)
        rby = jnp.minimum(ay2, by2)
        inter = jnp.maximum(rbx - ltx, zero) * jnp.maximum(rby - lty, zero)
        union = area_a + area_b - inter + eps
        rowg = base + lax.broadcasted_iota(jnp.int32, (_BLK, _N_PAD), 0)
        colg = lax.broadcasted_iota(jnp.int32, (_BLK, _N_PAD), 1)
        sup = (inter > thr * union) & (colg > rowg)
        slab_ref[:, :] = sup.astype(jnp.float32)
        cbx1 = bt_ref[0, 0:1, pl.ds(base, _BLK)]
        cby1 = bt_ref[0, 1:2, pl.ds(base, _BLK)]
        cbx2 = bt_ref[0, 2:3, pl.ds(base, _BLK)]
        cby2 = bt_ref[0, 3:4, pl.ds(base, _BLK)]
        carea = (cbx2 - cbx1) * (cby2 - cby1)
        dltx = jnp.maximum(ax1, cbx1)
        dlty = jnp.maximum(ay1, cby1)
        drbx = jnp.minimum(ax2, cbx2)
        drby = jnp.minimum(ay2, cby2)
        dint = jnp.maximum(drbx - dltx, zero) * jnp.maximum(drby - dlty, zero)
        duni = area_a + carea - dint + eps
        dsup = (dint > thr * duni) & tri_blk
        diag_ref[:, :] = dsup.astype(jnp.float32)
        sblk = supp_ref[0:1, pl.ds(base, _BLK)]
        for i in range(_BLK):
            row = diag_ref[i:i + 1, :]
            onehot = (jidx_blk == i).astype(jnp.float32)
            sflag = jnp.sum(sblk * onehot, axis=1, keepdims=True)
            gate = 1.0 - sflag
            gcol_ref[i:i + 1, :] = gate
            sblk = jnp.maximum(sblk, row * gate)
        gates = gcol_ref[:, :]
        acc = supp_ref[0:1, :]
        for j in range(_BLK // 16):
            part = slab_ref[j * 16:(j + 1) * 16, :] * gates[j * 16:(j + 1) * 16, :]
            acc = jnp.maximum(acc, jnp.max(part, axis=0, keepdims=True))
        supp_ref[0:1, :] = acc
        return 0

    lax.fori_loop(0, _NBLK, outer, 0)
    keep = supp_ref[0:1, :] < 0.5
    out_ref[0, 0:1, :] = jnp.where(keep, s_ref[0, 0:1, :], jnp.float32(-1.0))


def _run_decode(anchors_t, deltas_t, logits, levels, ih, iw):
    B = anchors_t.shape[0]
    spec4 = pl.BlockSpec((1, 4, _N_PAD), lambda b: (b, 0, 0))
    spec1 = pl.BlockSpec((1, 1, _N_PAD), lambda b: (b, 0, 0))
    specl = pl.BlockSpec((1, 1, _N_PAD), lambda b: (0, 0, 0))
    specs = pl.BlockSpec((1, 1), lambda b: (0, 0))
    return pl.pallas_call(
        _decode_kernel,
        grid=(B,),
        in_specs=[spec4, spec4, spec1, specl, specs, specs],
        out_specs=[spec4, spec4, spec1],
        out_shape=[
            jax.ShapeDtypeStruct((B, 4, _N_PAD), jnp.float32),
            jax.ShapeDtypeStruct((B, 4, _N_PAD), jnp.float32),
            jax.ShapeDtypeStruct((B, 1, _N_PAD), jnp.float32),
        ],
    )(anchors_t, deltas_t, logits, levels, ih, iw)


def _run_nms(b_st, b_cols, s_s):
    B = b_st.shape[0]
    spec4 = pl.BlockSpec((1, 4, _N_PAD), lambda b: (b, 0, 0))
    specc = pl.BlockSpec((1, _N_PAD, 4), lambda b: (b, 0, 0))
    spec1 = pl.BlockSpec((1, 1, _N_PAD), lambda b: (b, 0, 0))
    return pl.pallas_call(
        _nms_kernel,
        grid=(B,),
        in_specs=[spec4, specc, spec1],
        out_specs=spec1,
        out_shape=jax.ShapeDtypeStruct((B, 1, _N_PAD), jnp.float32),
        scratch_shapes=[
            pltpu.VMEM((_BLK, _N_PAD), jnp.float32),
            pltpu.VMEM((_BLK, _BLK), jnp.float32),
            pltpu.VMEM((_BLK, 1), jnp.float32),
            pltpu.VMEM((1, _N_PAD), jnp.float32),
        ],
    )(b_st, b_cols, s_s)


@jax.jit
def _pipeline(cls_outs, reg_outs, img_h, img_w):
    B = cls_outs[0].shape[0]
    lg_all, dl_all, an_all = [], [], []
    for lvl, (c, r) in enumerate(zip(cls_outs, reg_outs)):
        H, W = _FEAT_HW[lvl]
        n = H * W * _A
        k = _KS[lvl]
        logits = jnp.transpose(c, (0, 2, 3, 1)).reshape(B, n)
        rg = jnp.transpose(r.reshape(B, _A, 4, H, W), (0, 3, 4, 1, 2)).reshape(B, n, 4)
        vals, inds = lax.top_k(logits, k)
        dl = jnp.take_along_axis(rg, inds[:, :, None], axis=1)
        an = jnp.asarray(_ANCHORS[lvl])[inds]
        lg_all.append(vals)
        dl_all.append(dl)
        an_all.append(an)
    logits = jnp.concatenate(lg_all, axis=1)
    deltas = jnp.concatenate(dl_all, axis=1)
    anchors = jnp.concatenate(an_all, axis=1)
    pad = _N_PAD - _NTOT
    logits = jnp.pad(logits, ((0, 0), (0, pad)), constant_values=-1e30)
    deltas = jnp.pad(deltas, ((0, 0), (0, pad), (0, 0)))
    anchors = jnp.pad(anchors, ((0, 0), (0, pad), (0, 0)))
    anchors_t = jnp.transpose(anchors, (0, 2, 1))
    deltas_t = jnp.transpose(deltas, (0, 2, 1))
    logits3 = logits[:, None, :]
    levels3 = jnp.asarray(_LEVELS)[None, None, :]
    ih = jnp.asarray(img_h).astype(jnp.float32).reshape(1, 1)
    iw = jnp.asarray(img_w).astype(jnp.float32).reshape(1, 1)

    props, boff, scores = _run_decode(anchors_t, deltas_t, logits3, levels3, ih, iw)
    scores2d = scores[:, 0, :]

    order = jnp.argsort(-scores2d, axis=1)
    s_s = jnp.take_along_axis(scores2d, order, axis=1)
    p_st = jnp.take_along_axis(props, order[:, None, :], axis=2)
    b_st = jnp.take_along_axis(boff, order[:, None, :], axis=2)
    b_cols = jnp.transpose(b_st, (0, 2, 1))

    final = _run_nms(b_st, b_cols, s_s[:, None, :])[:, 0, :]

    vals, sel = lax.top_k(final, _MAX_PER_IMG)
    good = vals > 0.0
    p_cols = jnp.transpose(p_st, (0, 2, 1))
    sel_boxes = jnp.take_along_axis(p_cols, sel[:, :, None], axis=1)
    out_boxes = jnp.where(good[:, :, None], sel_boxes, 0.0)
    out_scores = jnp.where(good, vals, 0.0)
    return out_boxes, out_scores


def kernel(cls_out0, cls_out1, cls_out2, cls_out3, cls_out4,
           reg_out0, reg_out1, reg_out2, reg_out3, reg_out4, img_h, img_w):
    return _pipeline(
        [cls_out0, cls_out1, cls_out2, cls_out3, cls_out4],
        [reg_out0, reg_out1, reg_out2, reg_out3, reg_out4],
        img_h, img_w)


# speedup vs baseline: 19.9349x; 1.6057x over previous
import functools
import numpy as np
import jax
import jax.numpy as jnp
from jax import lax
from jax.experimental import pallas as pl
from jax.experimental.pallas import tpu as pltpu

_STRIDES = (4, 8, 16, 32, 64)
_SCALES = (8.0,)
_RATIOS = (0.5, 1.0, 2.0)
_A = len(_SCALES) * len(_RATIOS)
_PRE_NMS = 1000
_MAX_PER_IMG = 500
_NMS_THR = 0.7
_FEAT_HW = [(128, 128), (64, 64), (32, 32), (16, 16), (8, 8)]

_N_PAD = 4096
_BLK = 128
_NBLK = _N_PAD // _BLK


def _np_grid_anchors(H, W, stride):
    scales = np.asarray(_SCALES, np.float32)
    ratios = np.asarray(_RATIOS, np.float32)
    h_r = np.sqrt(ratios)
    w_r = (1.0 / h_r).astype(np.float32)
    ws = (np.float32(stride) * w_r[:, None] * scales[None, :]).reshape(-1)
    hs = (np.float32(stride) * h_r[:, None] * scales[None, :]).reshape(-1)
    base = np.stack([-0.5 * ws, -0.5 * hs, 0.5 * ws, 0.5 * hs], axis=1)
    xs = np.arange(W, dtype=np.float32) * np.float32(stride)
    ys = np.arange(H, dtype=np.float32) * np.float32(stride)
    sx, sy = np.meshgrid(xs, ys)
    shifts = np.stack([sx, sy, sx, sy], axis=-1).reshape(-1, 4)
    return (shifts[:, None, :] + base[None, :, :]).reshape(-1, 4).astype(np.float32)


_ANCHORS = [_np_grid_anchors(H, W, s) for (H, W), s in zip(_FEAT_HW, _STRIDES)]
_KS = [min(_PRE_NMS, H * W * _A) for (H, W) in _FEAT_HW]
_NTOT = sum(_KS)
_LEVELS = np.concatenate(
    [np.full((k,), float(l), np.float32) for l, k in enumerate(_KS)]
    + [np.zeros((_N_PAD - _NTOT,), np.float32)]
)


def _decode_kernel(a_ref, d_ref, lg_ref, lv_ref, ih_ref, iw_ref,
                   prop_ref, boff_ref, sc_ref):
    ax1 = a_ref[0, 0:1, :]
    ay1 = a_ref[0, 1:2, :]
    ax2 = a_ref[0, 2:3, :]
    ay2 = a_ref[0, 3:4, :]
    dx = d_ref[0, 0:1, :]
    dy = d_ref[0, 1:2, :]
    dw = d_ref[0, 2:3, :]
    dh = d_ref[0, 3:4, :]
    px = (ax1 + ax2) * 0.5
    py = (ay1 + ay2) * 0.5
    pw = ax2 - ax1
    ph = ay2 - ay1
    max_ratio = np.float32(np.log(1000.0 / 16.0))
    dw = jnp.clip(dw, -max_ratio, max_ratio)
    dh = jnp.clip(dh, -max_ratio, max_ratio)
    gx = px + pw * dx
    gy = py + ph * dy
    gw = pw * jnp.exp(dw)
    gh = ph * jnp.exp(dh)
    ihv = ih_ref[0:1, 0:1]
    iwv = iw_ref[0:1, 0:1]
    zero = jnp.float32(0.0)
    x1 = jnp.minimum(jnp.maximum(gx - 0.5 * gw, zero), iwv)
    y1 = jnp.minimum(jnp.maximum(gy - 0.5 * gh, zero), ihv)
    x2 = jnp.minimum(jnp.maximum(gx + 0.5 * gw, zero), iwv)
    y2 = jnp.minimum(jnp.maximum(gy + 0.5 * gh, zero), ihv)
    prop_ref[0, 0:1, :] = x1
    prop_ref[0, 1:2, :] = y1
    prop_ref[0, 2:3, :] = x2
    prop_ref[0, 3:4, :] = y2
    w = x2 - x1
    h = y2 - y1
    score = jax.nn.sigmoid(lg_ref[0, 0:1, :])
    score = jnp.where((w > zero) & (h > zero), score, jnp.float32(-1.0))
    sc_ref[0, 0:1, :] = score
    off = lv_ref[0, 0:1, :] * (jnp.maximum(ihv, iwv) + 1.0)
    boff_ref[0, 0:1, :] = x1 + off
    boff_ref[0, 1:2, :] = y1 + off
    boff_ref[0, 2:3, :] = x2 + off
    boff_ref[0, 3:4, :] = y2 + off


def _nms_kernel(bt_ref, bc_ref, s_ref, out_ref, slab_ref, supp_ref):
    bx1 = bt_ref[0, 0:1, :]
    by1 = bt_ref[0, 1:2, :]
    bx2 = bt_ref[0, 2:3, :]
    by2 = bt_ref[0, 3:4, :]
    area_b = (bx2 - bx1) * (by2 - by1)
    thr = jnp.float32(_NMS_THR)
    eps = jnp.float32(1e-9)
    zero = jnp.float32(0.0)
    supp_ref[:, :] = jnp.zeros((1, _N_PAD), jnp.float32)
    tri_blk = (lax.broadcasted_iota(jnp.int32, (_BLK, _BLK), 1)
               > lax.broadcasted_iota(jnp.int32, (_BLK, _BLK), 0))
    dnums = (((1,), (0,)), ((), ()))

    def outer(r, _):
        base = pl.multiple_of(r * _BLK, _BLK)
        ax1 = bc_ref[0, pl.ds(base, _BLK), 0:1]
        ay1 = bc_ref[0, pl.ds(base, _BLK), 1:2]
        ax2 = bc_ref[0, pl.ds(base, _BLK), 2:3]
        ay2 = bc_ref[0, pl.ds(base, _BLK), 3:4]
        area_a = (ax2 - ax1) * (ay2 - ay1)
        ltx = jnp.maximum(ax1, bx1)
        lty = jnp.maximum(ay1, by1)
        rbx = jnp.minimum(ax2, bx2)
        rby = jnp.minimum(ay2, by2)
        inter = jnp.maximum(rbx - ltx, zero) * jnp.maximum(rby - lty, zero)
        union = area_a + area_b - inter + eps
        rowg = base + lax.broadcasted_iota(jnp.int32, (_BLK, _N_PAD), 0)
        colg = lax.broadcasted_iota(jnp.int32, (_BLK, _N_PAD), 1)
        sup = (inter > thr * union) & (colg > rowg)
        slab_ref[:, :] = sup.astype(jnp.float32)
        cbx1 = bt_ref[0, 0:1, pl.ds(base, _BLK)]
        cby1 = bt_ref[0, 1:2, pl.ds(base, _BLK)]
        cbx2 = bt_ref[0, 2:3, pl.ds(base, _BLK)]
        cby2 = bt_ref[0, 3:4, pl.ds(base, _BLK)]
        carea = (cbx2 - cbx1) * (cby2 - cby1)
        dltx = jnp.maximum(ax1, cbx1)
        dlty = jnp.maximum(ay1, cby1)
        drbx = jnp.minimum(ax2, cbx2)
        drby = jnp.minimum(ay2, cby2)
        dint = jnp.maximum(drbx - dltx, zero) * jnp.maximum(drby - dlty, zero)
        duni = area_a + carea - dint + eps
        dmat = ((dint > thr * duni) & tri_blk).astype(jnp.float32)
        g0 = 1.0 - supp_ref[0:1, pl.ds(base, _BLK)]

        def fix_cond(c):
            k, kprev = c
            return jnp.any(k != kprev)

        def fix_body(c):
            k, _ = c
            s = lax.dot_general(k, dmat, dnums,
                                preferred_element_type=jnp.float32)
            knew = jnp.where(s > 0.5, zero, g0)
            return knew, k

        k, _ = lax.while_loop(fix_cond, fix_body,
                              (g0, jnp.full((1, _BLK), -1.0, jnp.float32)))
        s_all = lax.dot_general(k, slab_ref[:, :], dnums,
                                preferred_element_type=jnp.float32)
        supp_ref[0:1, :] = jnp.maximum(
            supp_ref[0:1, :], (s_all > 0.5).astype(jnp.float32))
        return 0

    lax.fori_loop(0, _NBLK, outer, 0)
    keep = supp_ref[0:1, :] < 0.5
    out_ref[0, 0:1, :] = jnp.where(keep, s_ref[0, 0:1, :], jnp.float32(-1.0))


def _run_decode(anchors_t, deltas_t, logits, levels, ih, iw):
    B = anchors_t.shape[0]
    spec4 = pl.BlockSpec((1, 4, _N_PAD), lambda b: (b, 0, 0))
    spec1 = pl.BlockSpec((1, 1, _N_PAD), lambda b: (b, 0, 0))
    specl = pl.BlockSpec((1, 1, _N_PAD), lambda b: (0, 0, 0))
    specs = pl.BlockSpec((1, 1), lambda b: (0, 0))
    return pl.pallas_call(
        _decode_kernel,
        grid=(B,),
        in_specs=[spec4, spec4, spec1, specl, specs, specs],
        out_specs=[spec4, spec4, spec1],
        out_shape=[
            jax.ShapeDtypeStruct((B, 4, _N_PAD), jnp.float32),
            jax.ShapeDtypeStruct((B, 4, _N_PAD), jnp.float32),
            jax.ShapeDtypeStruct((B, 1, _N_PAD), jnp.float32),
        ],
    )(anchors_t, deltas_t, logits, levels, ih, iw)


def _run_nms(b_st, b_cols, s_s):
    B = b_st.shape[0]
    spec4 = pl.BlockSpec((1, 4, _N_PAD), lambda b: (b, 0, 0))
    specc = pl.BlockSpec((1, _N_PAD, 4), lambda b: (b, 0, 0))
    spec1 = pl.BlockSpec((1, 1, _N_PAD), lambda b: (b, 0, 0))
    return pl.pallas_call(
        _nms_kernel,
        grid=(B,),
        in_specs=[spec4, specc, spec1],
        out_specs=spec1,
        out_shape=jax.ShapeDtypeStruct((B, 1, _N_PAD), jnp.float32),
        scratch_shapes=[
            pltpu.VMEM((_BLK, _N_PAD), jnp.float32),
            pltpu.VMEM((1, _N_PAD), jnp.float32),
        ],
    )(b_st, b_cols, s_s)


@jax.jit
def _pipeline(cls_outs, reg_outs, img_h, img_w):
    B = cls_outs[0].shape[0]
    lg_all, dl_all, an_all = [], [], []
    for lvl, (c, r) in enumerate(zip(cls_outs, reg_outs)):
        H, W = _FEAT_HW[lvl]
        n = H * W * _A
        k = _KS[lvl]
        logits = jnp.transpose(c, (0, 2, 3, 1)).reshape(B, n)
        rg = jnp.transpose(r.reshape(B, _A, 4, H, W), (0, 3, 4, 1, 2)).reshape(B, n, 4)
        vals, inds = lax.top_k(logits, k)
        dl = jnp.take_along_axis(rg, inds[:, :, None], axis=1)
        an = jnp.asarray(_ANCHORS[lvl])[inds]
        lg_all.append(vals)
        dl_all.append(dl)
        an_all.append(an)
    logits = jnp.concatenate(lg_all, axis=1)
    deltas = jnp.concatenate(dl_all, axis=1)
    anchors = jnp.concatenate(an_all, axis=1)
    pad = _N_PAD - _NTOT
    logits = jnp.pad(logits, ((0, 0), (0, pad)), constant_values=-1e30)
    deltas = jnp.pad(deltas, ((0, 0), (0, pad), (0, 0)))
    anchors = jnp.pad(anchors, ((0, 0), (0, pad), (0, 0)))
    anchors_t = jnp.transpose(anchors, (0, 2, 1))
    deltas_t = jnp.transpose(deltas, (0, 2, 1))
    logits3 = logits[:, None, :]
    levels3 = jnp.asarray(_LEVELS)[None, None, :]
    ih = jnp.asarray(img_h).astype(jnp.float32).reshape(1, 1)
    iw = jnp.asarray(img_w).astype(jnp.float32).reshape(1, 1)

    props, boff, scores = _run_decode(anchors_t, deltas_t, logits3, levels3, ih, iw)
    scores2d = scores[:, 0, :]

    order = jnp.argsort(-scores2d, axis=1)
    s_s = jnp.take_along_axis(scores2d, order, axis=1)
    p_st = jnp.take_along_axis(props, order[:, None, :], axis=2)
    b_st = jnp.take_along_axis(boff, order[:, None, :], axis=2)
    b_cols = jnp.transpose(b_st, (0, 2, 1))

    final = _run_nms(b_st, b_cols, s_s[:, None, :])[:, 0, :]

    vals, sel = lax.top_k(final, _MAX_PER_IMG)
    good = vals > 0.0
    p_cols = jnp.transpose(p_st, (0, 2, 1))
    sel_boxes = jnp.take_along_axis(p_cols, sel[:, :, None], axis=1)
    out_boxes = jnp.where(good[:, :, None], sel_boxes, 0.0)
    out_scores = jnp.where(good, vals, 0.0)
    return out_boxes, out_scores


def kernel(cls_out0, cls_out1, cls_out2, cls_out3, cls_out4,
           reg_out0, reg_out1, reg_out2, reg_out3, reg_out4, img_h, img_w):
    return _pipeline(
        [cls_out0, cls_out1, cls_out2, cls_out3, cls_out4],
        [reg_out0, reg_out1, reg_out2, reg_out3, reg_out4],
        img_h, img_w)
